# baseline (device time: 122299 ns/iter reference)
import jax
import jax.numpy as jnp
from jax import lax
from jax.experimental import pallas as pl
from jax.experimental.pallas import tpu as pltpu

N_DEV = 8
NP = 4
BLK = 64
STRIDE = 4


def kernel(x, Wq, K_ext, V_ext, Wo):
    B, Sq, E = x.shape
    _, S, H, D = K_ext.shape
    F = H * D
    Hh = H // 2
    Fh = Hh * D
    R = Sq // STRIDE
    NB = Sq // (STRIDE * BLK)

    def body(x_ref, wq_ref, k_ref, v_ref, wo_ref, o_ref,
             klo, vlo, khi, vhi, qbuf, acc, lsum, ctx,
             cw_s, cw_r, ccw_s, ccw_r, z_s, z_r):
        my = lax.axis_index("i")
        pp = lax.rem(my, NP)
        pl4 = my - pp
        opl4 = NP - pl4
        cwn = pl4 + lax.rem(pp + 1, NP)
        ccwn = pl4 + lax.rem(pp + 3, NP)
        ptn = opl4 + pp

        bufs = (klo, vlo, khi, vhi)

        barrier = pltpu.get_barrier_semaphore()
        for nbr in (cwn, ccwn, ptn):
            pl.semaphore_signal(
                barrier, inc=1,
                device_id=(nbr,), device_id_type=pl.DeviceIdType.MESH,
            )
        pl.semaphore_wait(barrier, 3)

        klo[my] = k_ref[:, :, :Fh]
        vlo[my] = v_ref[:, :, :Fh]
        khi[my] = k_ref[:, :, Fh:]
        vhi[my] = v_ref[:, :, Fh:]

        acc[...] = jnp.zeros_like(acc)
        lsum[...] = jnp.zeros_like(lsum)

        def rdma(bi, slot, sems, rsems, h, tgt):
            return pltpu.make_async_remote_copy(
                src_ref=bufs[bi].at[slot], dst_ref=bufs[bi].at[slot],
                send_sem=sems.at[bi, h], recv_sem=rsems.at[bi, h],
                device_id=(tgt,), device_id_type=pl.DeviceIdType.MESH,
            )

        def plane_hop(h, cw_slot, cw_bis, ccw_slot, ccw_bis):
            return (
                [rdma(bi, cw_slot, cw_s, cw_r, h, cwn) for bi in cw_bis]
                + [rdma(bi, ccw_slot, ccw_s, ccw_r, h, ccwn) for bi in ccw_bis]
            )

        def start(rs):
            for r_ in rs:
                r_.start()

        def wait(rs):
            for r_ in rs:
                r_.wait()

        def process(kbuf, vbuf, slot, half):
            def rb_body(i, _):
                r = i // B
                b = lax.rem(i, B)
                kc = jnp.concatenate(
                    [kbuf[slot, b, pl.ds((r + STRIDE * a) * BLK, BLK), :]
                     for a in range(NB)], axis=0)
                vc = jnp.concatenate(
                    [vbuf[slot, b, pl.ds((r + STRIDE * a) * BLK, BLK), :]
                     for a in range(NB)], axis=0)
                q = qbuf[r, b, :, pl.ds(half * Fh, Fh)]
                s = jnp.einsum(
                    "ihd,jhd->hij",
                    q.reshape(R, Hh, D),
                    kc.reshape(NB * BLK, Hh, D),
                    preferred_element_type=jnp.float32,
                ) * 0.125
                e = jnp.exp(s)
                lsum[r, b, pl.ds(half * Hh, Hh), :] = (
                    lsum[r, b, pl.ds(half * Hh, Hh), :] + e.sum(axis=-1)
                )
                c = jnp.einsum(
                    "hij,jhd->ihd",
                    e.astype(jnp.bfloat16),
                    vc.reshape(NB * BLK, Hh, D),
                    preferred_element_type=jnp.float32,
                ).reshape(R, Fh)
                acc[r, b, :, pl.ds(half * Fh, Fh)] = (
                    acc[r, b, :, pl.ds(half * Fh, Fh)] + c
                )
                return 0

            lax.fori_loop(0, STRIDE * B, rb_body, 0)

        LO, HI = (0, 1), (2, 3)

        z0 = [rdma(bi, my, z_s, z_r, 0, ptn) for bi in range(4)]
        h0 = plane_hop(0, my, LO, my, HI)
        start(z0)
        start(h0)

        for r in range(STRIDE):
            for b in range(B):
                xs = jnp.concatenate(
                    [x_ref[b, (r + STRIDE * a) * BLK:
                           (r + STRIDE * a + 1) * BLK, :] for a in range(NB)],
                    axis=0,
                )
                qbuf[r, b] = jnp.dot(
                    xs, wq_ref[...], preferred_element_type=jnp.float32
                ).astype(jnp.bfloat16)

        process(klo, vlo, my, 0)
        process(khi, vhi, my, 1)
        wait(h0)

        z1 = ([rdma(bi, ccwn, z_s, z_r, 1, ptn) for bi in LO]
              + [rdma(bi, cwn, z_s, z_r, 1, ptn) for bi in HI])
        h1 = plane_hop(1, ccwn, LO, cwn, HI)
        start(z1)
        start(h1)
        process(klo, vlo, ccwn, 0)
        process(khi, vhi, cwn, 1)
        wait(h1)

        far = pl4 + lax.rem(pp + 2, NP)
        h2 = plane_hop(2, far, LO, far, HI)
        start(h2)
        process(klo, vlo, far, 0)
        process(khi, vhi, far, 1)
        wait(h2)
        wait(z0)

        hA = plane_hop(3, ptn, HI, ptn, LO)
        start(hA)
        process(klo, vlo, cwn, 0)
        process(khi, vhi, ccwn, 1)
        process(klo, vlo, ptn, 0)
        process(khi, vhi, ptn, 1)
        wait(hA)
        wait(z1)

        z_prev = opl4 + lax.rem(pp + 3, NP)
        z_next = opl4 + lax.rem(pp + 1, NP)
        hB = plane_hop(4, z_prev, LO, z_next, HI)
        start(hB)
        process(klo, vlo, z_prev, 0)
        process(khi, vhi, z_prev, 1)
        process(klo, vlo, z_next, 0)
        process(khi, vhi, z_next, 1)
        wait(hB)

        z_far = opl4 + lax.rem(pp + 2, NP)
        process(klo, vlo, z_far, 0)
        process(khi, vhi, z_far, 1)

        for r in range(STRIDE):
            a_r = acc[r]
            l_r = lsum[r]
            ctxr = a_r.reshape(B, R, H, D) / jnp.transpose(
                l_r, (0, 2, 1)
            )[:, :, :, None]
            ctxr = ctxr.reshape(B, R, F).astype(jnp.bfloat16)
            for a in range(NB):
                ctx[:, pl.ds((STRIDE * a + r) * BLK, BLK), :] = (
                    ctxr[:, a * BLK:(a + 1) * BLK, :]
                )

        for b in range(B):
            o_ref[b] = jnp.dot(
                ctx[b], wo_ref[...], preferred_element_type=jnp.float32
            )

    f32 = jnp.float32
    bf16 = jnp.bfloat16
    return pl.pallas_call(
        body,
        out_shape=jax.ShapeDtypeStruct((B, Sq, E), f32),
        in_specs=[pl.BlockSpec(memory_space=pltpu.VMEM)] * 5,
        out_specs=pl.BlockSpec(memory_space=pltpu.VMEM),
        scratch_shapes=[
            pltpu.VMEM((N_DEV, B, S, Fh), bf16),
            pltpu.VMEM((N_DEV, B, S, Fh), bf16),
            pltpu.VMEM((N_DEV, B, S, Fh), bf16),
            pltpu.VMEM((N_DEV, B, S, Fh), bf16),
            pltpu.VMEM((STRIDE, B, R, F), bf16),
            pltpu.VMEM((STRIDE, B, R, F), f32),
            pltpu.VMEM((STRIDE, B, H, R), f32),
            pltpu.VMEM((B, Sq, F), bf16),
            pltpu.SemaphoreType.DMA((4, 5)),
            pltpu.SemaphoreType.DMA((4, 5)),
            pltpu.SemaphoreType.DMA((4, 5)),
            pltpu.SemaphoreType.DMA((4, 5)),
            pltpu.SemaphoreType.DMA((4, 2)),
            pltpu.SemaphoreType.DMA((4, 2)),
        ],
        compiler_params=pltpu.CompilerParams(
            collective_id=0, vmem_limit_bytes=100 * 1024 * 1024,
        ),
    )(x.astype(bf16), Wq.astype(bf16),
      K_ext.reshape(B, S, F).astype(bf16),
      V_ext.reshape(B, S, F).astype(bf16), Wo.astype(bf16))


# device time: 118654 ns/iter; 1.0307x vs baseline; 1.0307x over previous
import jax
import jax.numpy as jnp
from jax import lax
from jax.experimental import pallas as pl
from jax.experimental.pallas import tpu as pltpu

N_DEV = 8
NP = 4
BLK = 64
STRIDE = 4


def kernel(x, Wq, K_ext, V_ext, Wo):
    B, Sq, E = x.shape
    _, S, H, D = K_ext.shape
    F = H * D
    Hh = H // 2
    Fh = Hh * D
    R = Sq // STRIDE
    NB = Sq // (STRIDE * BLK)

    def body(x_ref, wq_ref, k_ref, v_ref, wo_ref, o_ref,
             klo, vlo, khi, vhi, qbuf, acc, lsum, ctx,
             cw_s, cw_r, ccw_s, ccw_r, z_s, z_r):
        my = lax.axis_index("i")
        pp = lax.rem(my, NP)
        pl4 = my - pp
        opl4 = NP - pl4
        cwn = pl4 + lax.rem(pp + 1, NP)
        ccwn = pl4 + lax.rem(pp + 3, NP)
        ptn = opl4 + pp

        bufs = (klo, vlo, khi, vhi)

        barrier = pltpu.get_barrier_semaphore()
        for nbr in (cwn, ccwn, ptn):
            pl.semaphore_signal(
                barrier, inc=1,
                device_id=(nbr,), device_id_type=pl.DeviceIdType.MESH,
            )
        pl.semaphore_wait(barrier, 3)

        klo[my] = k_ref[:, :, :Fh]
        vlo[my] = v_ref[:, :, :Fh]
        khi[my] = k_ref[:, :, Fh:]
        vhi[my] = v_ref[:, :, Fh:]

        acc[...] = jnp.zeros_like(acc)
        lsum[...] = jnp.zeros_like(lsum)

        def rdma(bi, slot, sems, rsems, h, tgt):
            return pltpu.make_async_remote_copy(
                src_ref=bufs[bi].at[slot], dst_ref=bufs[bi].at[slot],
                send_sem=sems.at[bi, h], recv_sem=rsems.at[bi, h],
                device_id=(tgt,), device_id_type=pl.DeviceIdType.MESH,
            )

        def plane_hop(h, cw_slot, cw_bis, ccw_slot, ccw_bis):
            return (
                [rdma(bi, cw_slot, cw_s, cw_r, h, cwn) for bi in cw_bis]
                + [rdma(bi, ccw_slot, ccw_s, ccw_r, h, ccwn) for bi in ccw_bis]
            )

        def start(rs):
            for r_ in rs:
                r_.start()

        def wait(rs):
            for r_ in rs:
                r_.wait()

        def process(kbuf, vbuf, slot, half):
            def rb_body(i, _):
                r = i // B
                b = lax.rem(i, B)
                kc = jnp.concatenate(
                    [kbuf[slot, b, pl.ds((r + STRIDE * a) * BLK, BLK), :]
                     for a in range(NB)], axis=0)
                vc = jnp.concatenate(
                    [vbuf[slot, b, pl.ds((r + STRIDE * a) * BLK, BLK), :]
                     for a in range(NB)], axis=0)
                q = qbuf[r, b, :, pl.ds(half * Fh, Fh)]
                s = jnp.einsum(
                    "ihd,jhd->hij",
                    q.reshape(R, Hh, D),
                    kc.reshape(NB * BLK, Hh, D),
                    preferred_element_type=jnp.float32,
                ) * 0.125
                e = jnp.exp(s)
                lsum[r, b, pl.ds(half * Hh, Hh), :] = (
                    lsum[r, b, pl.ds(half * Hh, Hh), :] + e.sum(axis=-1)
                )
                c = jnp.einsum(
                    "hij,jhd->ihd",
                    e.astype(jnp.bfloat16),
                    vc.reshape(NB * BLK, Hh, D),
                    preferred_element_type=jnp.float32,
                ).reshape(R, Fh)
                acc[r, b, :, pl.ds(half * Fh, Fh)] = (
                    acc[r, b, :, pl.ds(half * Fh, Fh)] + c
                )
                return 0

            lax.fori_loop(0, STRIDE * B, rb_body, 0)

        LO, HI = (0, 1), (2, 3)

        z0 = [rdma(bi, my, z_s, z_r, 0, ptn) for bi in range(4)]
        h0 = plane_hop(0, my, LO, my, HI)
        start(z0)
        start(h0)

        for r in range(STRIDE):
            for b in range(B):
                xs = jnp.concatenate(
                    [x_ref[b, (r + STRIDE * a) * BLK:
                           (r + STRIDE * a + 1) * BLK, :] for a in range(NB)],
                    axis=0,
                )
                qbuf[r, b] = jnp.dot(
                    xs, wq_ref[...], preferred_element_type=jnp.float32
                ).astype(jnp.bfloat16)

        process(klo, vlo, my, 0)
        process(khi, vhi, my, 1)
        wait(h0)

        z1 = ([rdma(bi, ccwn, z_s, z_r, 1, ptn) for bi in LO]
              + [rdma(bi, cwn, z_s, z_r, 1, ptn) for bi in HI])
        h1 = plane_hop(1, ccwn, LO, cwn, HI)
        start(z1)
        start(h1)
        process(klo, vlo, ccwn, 0)
        process(khi, vhi, cwn, 1)
        wait(h1)

        far = pl4 + lax.rem(pp + 2, NP)
        h2 = plane_hop(2, far, LO, far, HI)
        start(h2)
        process(klo, vlo, far, 0)
        process(khi, vhi, far, 1)
        wait(h2)
        wait(z0)

        hA = plane_hop(3, ptn, HI, ptn, LO)
        start(hA)
        process(klo, vlo, cwn, 0)
        process(khi, vhi, ccwn, 1)
        process(klo, vlo, ptn, 0)
        process(khi, vhi, ptn, 1)
        wait(hA)
        wait(z1)

        z_prev = opl4 + lax.rem(pp + 3, NP)
        z_next = opl4 + lax.rem(pp + 1, NP)
        hB = plane_hop(4, z_prev, LO, z_next, HI)
        start(hB)
        process(klo, vlo, z_prev, 0)
        process(khi, vhi, z_prev, 1)
        process(klo, vlo, z_next, 0)
        process(khi, vhi, z_next, 1)
        wait(hB)

        z_far = opl4 + lax.rem(pp + 2, NP)
        process(klo, vlo, z_far, 0)
        process(khi, vhi, z_far, 1)

        for r in range(STRIDE):
            a_r = acc[r]
            l_r = lsum[r]
            ctxr = a_r.reshape(B, R, H, D) / jnp.transpose(
                l_r, (0, 2, 1)
            )[:, :, :, None]
            ctxr = ctxr.reshape(B, R, F)
            for a in range(NB):
                ctx[:, pl.ds((STRIDE * a + r) * BLK, BLK), :] = (
                    ctxr[:, a * BLK:(a + 1) * BLK, :]
                )

        for b in range(B):
            o_ref[b] = jnp.dot(
                ctx[b], wo_ref[...], preferred_element_type=jnp.float32
            )

    f32 = jnp.float32
    bf16 = jnp.bfloat16
    return pl.pallas_call(
        body,
        out_shape=jax.ShapeDtypeStruct((B, Sq, E), f32),
        in_specs=[pl.BlockSpec(memory_space=pltpu.VMEM)] * 5,
        out_specs=pl.BlockSpec(memory_space=pltpu.VMEM),
        scratch_shapes=[
            pltpu.VMEM((N_DEV, B, S, Fh), bf16),
            pltpu.VMEM((N_DEV, B, S, Fh), bf16),
            pltpu.VMEM((N_DEV, B, S, Fh), bf16),
            pltpu.VMEM((N_DEV, B, S, Fh), bf16),
            pltpu.VMEM((STRIDE, B, R, F), bf16),
            pltpu.VMEM((STRIDE, B, R, F), f32),
            pltpu.VMEM((STRIDE, B, H, R), f32),
            pltpu.VMEM((B, Sq, F), f32),
            pltpu.SemaphoreType.DMA((4, 5)),
            pltpu.SemaphoreType.DMA((4, 5)),
            pltpu.SemaphoreType.DMA((4, 5)),
            pltpu.SemaphoreType.DMA((4, 5)),
            pltpu.SemaphoreType.DMA((4, 2)),
            pltpu.SemaphoreType.DMA((4, 2)),
        ],
        compiler_params=pltpu.CompilerParams(
            collective_id=0, vmem_limit_bytes=100 * 1024 * 1024,
        ),
    )(x, Wq,
      K_ext.reshape(B, S, F).astype(bf16),
      V_ext.reshape(B, S, F).astype(bf16), Wo)
